# baseline (speedup 1.0000x reference)
import functools

import jax
import jax.numpy as jnp
import numpy as np
from jax import lax
from jax.experimental import pallas as pl
from jax.experimental.pallas import tpu as pltpu
from jax.experimental.pallas import tpu_sc as plsc

_N = 10000
_E = 320000
_D = 128

_NC = 2
_NS = 16
_NW = _NC * _NS
_EPT = _E // _NW
_C = 80
_NCH = _EPT // _C
_RB = 80
_NRB = _N // _RB
_ZB = 8
_NI = 3
_ND = 2
_UN = 6

_S_LO = np.zeros((128, 64), np.float32)
_S_HI = np.zeros((128, 64), np.float32)
for _g in range(4):
    for _i in range(16):
        _S_LO[32 * _g + _i, 16 * _g + _i] = 1.0
        _S_HI[32 * _g + 16 + _i, 16 * _g + _i] = 1.0

_SC_MESH = plsc.VectorSubcoreMesh(core_axis_name="c", subcore_axis_name="s")


@functools.partial(
    pl.kernel,
    out_type=jax.ShapeDtypeStruct((2 * _N, _D), jnp.float32),
    mesh=_SC_MESH,
    scratch_types=(
        [pltpu.VMEM((_C,), jnp.int32)] * (2 * _NI)
        + [pltpu.VMEM((_C, _D), jnp.float32)] * _ND
        + [pltpu.VMEM((_C // 2, _D), jnp.int32)] * _ND
        + [pltpu.VMEM((_ZB, _D), jnp.float32)]
        + [pltpu.VMEM_SHARED((_N, _D), jnp.float32)]
        + [pltpu.SemaphoreType.DMA] * (_NI + 2 * _ND)
    ),
)
def _sc_edge(h_hbm, e_hbm, src_hbm, dst_hbm, out_hbm,
             si0, si1, si2, di0, di1, di2, r0, r1, e0, e1, zbuf, agg,
             i0, i1, i2, d0, d1, s0, s1):
    sidx = (si0, si1, si2)
    didx = (di0, di1, di2)
    rows = (r0, r1)
    ebuf = (e0, e1)
    isem = (i0, i1, i2)
    dsem = (d0, d1)
    ssem = (s0, s1)
    c = lax.axis_index("c")
    s = lax.axis_index("s")
    wid = s * _NC + c

    blo = s * _NRB // _NS
    bhi = (s + 1) * _NRB // _NS

    def _z(i, t):
        for j in range(_D // 16):
            zbuf[i, pl.ds(j * 16, 16)] = jnp.zeros((16,), jnp.float32)
        return t
    lax.fori_loop(0, _ZB, _z, 0)

    def _zb(b, t):
        pltpu.sync_copy(zbuf, agg.at[pl.ds(b * _ZB, _ZB)])
        return t
    lax.fori_loop(blo * (_RB // _ZB), bhi * (_RB // _ZB), _zb, 0)

    def _stage_idx(n, bi):
        base = wid * _EPT + n * _C
        pltpu.async_copy(src_hbm.at[pl.ds(base, _C)], sidx[bi], isem[bi])
        pltpu.async_copy(dst_hbm.at[pl.ds(base, _C)], didx[bi], isem[bi])

    def _wait_idx(n, bi):
        base = wid * _EPT + n * _C
        pltpu.make_async_copy(src_hbm.at[pl.ds(base, _C)], sidx[bi],
                              isem[bi]).wait()
        pltpu.make_async_copy(dst_hbm.at[pl.ds(base, _C)], didx[bi],
                              isem[bi]).wait()

    def _stage_data(n, b, bi):
        ebase = wid * (_EPT // 2) + n * (_C // 2)
        pltpu.async_copy(h_hbm.at[sidx[bi]], rows[b], dsem[b])
        pltpu.async_copy(e_hbm.at[pl.ds(ebase, _C // 2)], ebuf[b], dsem[b])

    def _wait_data(n, b, bi):
        ebase = wid * (_EPT // 2) + n * (_C // 2)
        pltpu.make_async_copy(h_hbm.at[sidx[bi]], rows[b], dsem[b]).wait()
        pltpu.make_async_copy(e_hbm.at[pl.ds(ebase, _C // 2)], ebuf[b],
                              dsem[b]).wait()

    def _wait_scatter(b, bi):
        pltpu.make_async_copy(rows[b], agg.at[didx[bi]], ssem[b]).wait()

    _mask = jnp.int32(-65536)

    def _process(n, b, bi):
        _wait_data(n, b, bi)

        def _v(q, u):
            for p in range(2):
                r = q * 2 + p
                for g in range(4):
                    w = ebuf[b][q, pl.ds(64 * p + 16 * g, 16)]
                    le = lax.bitcast_convert_type(w << 16, jnp.float32)
                    he = lax.bitcast_convert_type(w & _mask, jnp.float32)
                    slo = pl.ds(32 * g, 16)
                    shi = pl.ds(32 * g + 16, 16)
                    rows[b][r, slo] = jnp.maximum(rows[b][r, slo] + le, 0.0)
                    rows[b][r, shi] = jnp.maximum(rows[b][r, shi] + he, 0.0)
            return u
        lax.fori_loop(0, _C // 2, _v, 0)
        pltpu.async_copy(rows[b], agg.at[didx[bi]], ssem[b], add=True)

    plsc.subcore_barrier()

    _stage_idx(0, 0)
    _stage_idx(1, 1)
    _wait_idx(0, 0)
    _stage_data(0, 0, 0)

    def _step(n, db):
        @pl.when(n + 2 < _NCH)
        def _():
            _stage_idx(n + 2, (db + 2) % _NI)

        @pl.when((n + 1 < _NCH) & (n >= 1))
        def _():
            _wait_scatter((db + 1) % _ND, (db + 1) % _NI)

        @pl.when(n + 1 < _NCH)
        def _():
            _wait_idx(n + 1, (db + 1) % _NI)
            _stage_data(n + 1, (db + 1) % _ND, (db + 1) % _NI)

        _process(n, db % _ND, db % _NI)

    def _main(i, t):
        for db in range(_UN):
            _step(i * _UN + db, db)
        return t
    lax.fori_loop(0, _NCH // _UN, _main, 0)
    for n in range(_NCH - _NCH % _UN, _NCH):
        _step(n, n % _UN)
    for b in range(_ND):
        _wait_scatter(b, b)

    plsc.subcore_barrier()

    def _wb(b, t):
        pltpu.sync_copy(agg.at[pl.ds(b * _RB, _RB)],
                        out_hbm.at[pl.ds(c * _N + b * _RB, _RB)])
        return t
    lax.fori_loop(blo, bhi, _wb, 0)


def _mm_bias_body(a_ref, w_ref, b_ref, o_ref):
    o_ref[...] = (jnp.dot(a_ref[...], w_ref[...],
                          preferred_element_type=jnp.float32) + b_ref[...])


def _mm_bias(a, w, b, blk):
    m, k = a.shape
    n = w.shape[1]
    return pl.pallas_call(
        _mm_bias_body,
        grid=(m // blk,),
        in_specs=[pl.BlockSpec((blk, k), lambda i: (i, 0)),
                  pl.BlockSpec((k, n), lambda i: (0, 0)),
                  pl.BlockSpec((1, n), lambda i: (0, 0))],
        out_specs=pl.BlockSpec((blk, n), lambda i: (i, 0)),
        out_shape=jax.ShapeDtypeStruct((m, n), jnp.float32),
    )(a, w, b.reshape(1, n))


def _pack_cols(t, sl, sh):
    tl = jnp.dot(t, sl, preferred_element_type=jnp.float32)
    th = jnp.dot(t, sh, preferred_element_type=jnp.float32)
    tl = tl.astype(jnp.bfloat16).astype(jnp.float32)
    th = th.astype(jnp.bfloat16).astype(jnp.float32)
    lo = lax.bitcast_convert_type(tl, jnp.uint32) >> 16
    hi = lax.bitcast_convert_type(th, jnp.uint32)
    return lax.bitcast_convert_type(lo | hi, jnp.int32)


def _epack_body(a_ref, w_ref, b_ref, sl_ref, sh_ref, o_ref):
    w = w_ref[...]
    bb = b_ref[...]
    t0 = jnp.dot(a_ref[:, 0:16], w, preferred_element_type=jnp.float32) + bb
    t1 = jnp.dot(a_ref[:, 16:32], w, preferred_element_type=jnp.float32) + bb
    p0 = _pack_cols(t0, sl_ref[...], sh_ref[...])
    p1 = _pack_cols(t1, sl_ref[...], sh_ref[...])
    o_ref[...] = jnp.concatenate([p0, p1], axis=1)


def _epack(a2, w, b, slm, shm, blk):
    m = a2.shape[0]
    n = w.shape[1]
    full = lambda i: (0, 0)
    return pl.pallas_call(
        _epack_body,
        grid=(m // blk,),
        in_specs=[pl.BlockSpec((blk, 32), lambda i: (i, 0)),
                  pl.BlockSpec((16, n), full),
                  pl.BlockSpec((1, n), full),
                  pl.BlockSpec((n, n // 2), full),
                  pl.BlockSpec((n, n // 2), full)],
        out_specs=pl.BlockSpec((blk, n), lambda i: (i, 0)),
        out_shape=jax.ShapeDtypeStruct((m, n), jnp.int32),
    )(a2, w, b.reshape(1, n), slm, shm)


def _mlp_body(h_ref, p_ref, w1_ref, b1_ref, w2_ref, b2_ref, gs_ref, bt_ref,
              eps_ref, o_ref):
    h = h_ref[...]
    a = h * (1.0 + eps_ref[0, 0]) + p_ref[0] + p_ref[1]
    t = jnp.maximum(jnp.dot(a, w1_ref[...],
                            preferred_element_type=jnp.float32) + b1_ref[...],
                    0.0)
    t = jnp.dot(t, w2_ref[...], preferred_element_type=jnp.float32) + b2_ref[...]
    t = t * gs_ref[...] + bt_ref[...]
    o_ref[...] = jnp.maximum(t, 0.0) + h


def _mlp(h, parts, w1, b1, w2, b2, gscale, beta, eps, blk=2000):
    full = lambda i: (0, 0)
    return pl.pallas_call(
        _mlp_body,
        grid=(_N // blk,),
        in_specs=[pl.BlockSpec((blk, _D), lambda i: (i, 0)),
                  pl.BlockSpec((2, blk, _D), lambda i: (0, i, 0)),
                  pl.BlockSpec((_D, _D), full),
                  pl.BlockSpec((1, _D), full),
                  pl.BlockSpec((_D, _D), full),
                  pl.BlockSpec((1, _D), full),
                  pl.BlockSpec((1, _D), full),
                  pl.BlockSpec((1, _D), full),
                  pl.BlockSpec((1, 1), full)],
        out_specs=pl.BlockSpec((blk, _D), lambda i: (i, 0)),
        out_shape=jax.ShapeDtypeStruct((_N, _D), jnp.float32),
    )(h, parts, w1, b1.reshape(1, _D), w2, b2.reshape(1, _D),
      gscale.reshape(1, _D), beta.reshape(1, _D), eps.reshape(1, 1))


def _head_body(h_ref, b_ref, gx_ref, wg_ref, bg_ref, wc_ref, bc_ref, o_ref):
    g_iota = lax.broadcasted_iota(jnp.int32, (1, 16), 1)
    oh = (b_ref[...] == g_iota).astype(jnp.float32)
    dn = (((0,), (0,)), ((), ()))
    sums = lax.dot_general(oh, h_ref[...], dn,
                           preferred_element_type=jnp.float32)
    ones = jnp.ones((_N, 1), jnp.float32)
    counts = lax.dot_general(oh, ones, dn,
                             preferred_element_type=jnp.float32)
    pooled = sums / jnp.maximum(counts, 1.0)
    g = jnp.maximum(jnp.dot(gx_ref[...], wg_ref[...],
                            preferred_element_type=jnp.float32) + bg_ref[...],
                    0.0)
    out = (jnp.dot(pooled, wc_ref[0:_D, :],
                   preferred_element_type=jnp.float32)
           + jnp.dot(g, wc_ref[_D:_D + 32, :],
                     preferred_element_type=jnp.float32)
           + bc_ref[...])
    o_ref[...] = out


def _head(h, batch2d, gx, wg, bg, wc, bc):
    return pl.pallas_call(
        _head_body,
        out_shape=jax.ShapeDtypeStruct((16, _D), jnp.float32),
    )(h, batch2d, gx, wg, bg.reshape(1, -1), wc, bc.reshape(1, -1))


def kernel(x, edge_index, edge_attr, batch, global_x, params):
    src = edge_index[0]
    dst = edge_index[1]
    slm = jnp.asarray(_S_LO)
    shm = jnp.asarray(_S_HI)
    h = _mm_bias(x, params['W_in'], params['b_in'], 2000)
    e_pk = _epack(edge_attr.reshape(_E // 2, 32), params['W_e'],
                  params['b_e'], slm, shm, 8000)
    bn_scale = 1.0 / jnp.sqrt(jnp.float32(1.0 + 1e-5))
    for i in range(3):
        p = params['layers'][i]
        parts = _sc_edge(h, e_pk, src, dst).reshape(2, _N, _D)
        h = _mlp(h, parts, p['W1'], p['b1'], p['W2'], p['b2'],
                 p['gamma'] * bn_scale, p['beta'], p['eps'])
    return _head(h, batch.reshape(_N, 1), global_x,
                 params['Wg'], params['bg'], params['Wc'], params['bc'])


# speedup vs baseline: 1.3059x; 1.3059x over previous
import functools

import jax
import jax.numpy as jnp
from jax import lax
from jax.experimental import pallas as pl
from jax.experimental.pallas import tpu as pltpu
from jax.experimental.pallas import tpu_sc as plsc

_N = 10000
_E = 320000
_D = 128

_NC = 2
_NS = 16
_NW = _NC * _NS
_EPT = _E // _NW
_C = 40
_NCH = _EPT // _C
_RB = 80
_NRB = _N // _RB
_ZB = 8
_NI = 3
_ND = 2
_UN = 6

_SC_MESH = plsc.VectorSubcoreMesh(core_axis_name="c", subcore_axis_name="s")


@functools.partial(
    pl.kernel,
    out_type=jax.ShapeDtypeStruct((2 * _N, _D), jnp.float32),
    mesh=_SC_MESH,
    scratch_types=(
        [pltpu.VMEM((_C,), jnp.int32)] * (2 * _NI)
        + [pltpu.VMEM((_C, _D), jnp.float32)] * _ND
        + [pltpu.VMEM((_C, _D), jnp.float32)] * _ND
        + [pltpu.VMEM((_ZB, _D), jnp.float32)]
        + [pltpu.VMEM_SHARED((_N, _D), jnp.float32)]
        + [pltpu.SemaphoreType.DMA] * (_NI + 2 * _ND)
    ),
)
def _sc_edge(h_hbm, e_hbm, src_hbm, dst_hbm, out_hbm,
             si0, si1, si2, di0, di1, di2, r0, r1, e0, e1, zbuf, agg,
             i0, i1, i2, d0, d1, s0, s1):
    sidx = (si0, si1, si2)
    didx = (di0, di1, di2)
    rows = (r0, r1)
    ebuf = (e0, e1)
    isem = (i0, i1, i2)
    dsem = (d0, d1)
    ssem = (s0, s1)
    c = lax.axis_index("c")
    s = lax.axis_index("s")
    wid = s * _NC + c

    blo = s * _NRB // _NS
    bhi = (s + 1) * _NRB // _NS

    def _z(i, t):
        for j in range(_D // 16):
            zbuf[i, pl.ds(j * 16, 16)] = jnp.zeros((16,), jnp.float32)
        return t
    lax.fori_loop(0, _ZB, _z, 0)

    def _zb(b, t):
        pltpu.sync_copy(zbuf, agg.at[pl.ds(b * _ZB, _ZB)])
        return t
    lax.fori_loop(blo * (_RB // _ZB), bhi * (_RB // _ZB), _zb, 0)

    def _stage_idx(n, bi):
        base = wid * _EPT + n * _C
        pltpu.async_copy(src_hbm.at[pl.ds(base, _C)], sidx[bi], isem[bi])
        pltpu.async_copy(dst_hbm.at[pl.ds(base, _C)], didx[bi], isem[bi])

    def _wait_idx(n, bi):
        base = wid * _EPT + n * _C
        pltpu.make_async_copy(src_hbm.at[pl.ds(base, _C)], sidx[bi],
                              isem[bi]).wait()
        pltpu.make_async_copy(dst_hbm.at[pl.ds(base, _C)], didx[bi],
                              isem[bi]).wait()

    def _stage_data(n, b, bi):
        base = wid * _EPT + n * _C
        pltpu.async_copy(h_hbm.at[sidx[bi]], rows[b], dsem[b])
        pltpu.async_copy(e_hbm.at[pl.ds(base, _C)], ebuf[b], dsem[b])

    def _wait_data(n, b, bi):
        base = wid * _EPT + n * _C
        pltpu.make_async_copy(h_hbm.at[sidx[bi]], rows[b], dsem[b]).wait()
        pltpu.make_async_copy(e_hbm.at[pl.ds(base, _C)], ebuf[b],
                              dsem[b]).wait()

    def _wait_scatter(b, bi):
        pltpu.make_async_copy(rows[b], agg.at[didx[bi]], ssem[b]).wait()

    def _process(n, b, bi):
        _wait_data(n, b, bi)

        def _v(r, u):
            for j in range(_D // 16):
                sl = pl.ds(j * 16, 16)
                rows[b][r, sl] = jnp.maximum(rows[b][r, sl] + ebuf[b][r, sl],
                                             0.0)
            return u
        lax.fori_loop(0, _C, _v, 0)
        pltpu.async_copy(rows[b], agg.at[didx[bi]], ssem[b], add=True)

    plsc.subcore_barrier()

    _stage_idx(0, 0)
    _stage_idx(1, 1)
    _wait_idx(0, 0)
    _stage_data(0, 0, 0)

    def _step(n, db):
        @pl.when(n + 2 < _NCH)
        def _():
            _stage_idx(n + 2, (db + 2) % _NI)

        @pl.when((n + 1 < _NCH) & (n >= 1))
        def _():
            _wait_scatter((db + 1) % _ND, (db + 1) % _NI)

        @pl.when(n + 1 < _NCH)
        def _():
            _wait_idx(n + 1, (db + 1) % _NI)
            _stage_data(n + 1, (db + 1) % _ND, (db + 1) % _NI)

        _process(n, db % _ND, db % _NI)

    def _main(i, t):
        for db in range(_UN):
            _step(i * _UN + db, db)
        return t
    lax.fori_loop(0, _NCH // _UN, _main, 0)
    for n in range(_NCH - _NCH % _UN, _NCH):
        _step(n, n % _UN)
    for b in range(_ND):
        _wait_scatter(b, b)

    plsc.subcore_barrier()

    def _wb(b, t):
        pltpu.sync_copy(agg.at[pl.ds(b * _RB, _RB)],
                        out_hbm.at[pl.ds(c * _N + b * _RB, _RB)])
        return t
    lax.fori_loop(blo, bhi, _wb, 0)


def _mm_bias_body(a_ref, w_ref, b_ref, o_ref):
    o_ref[...] = (jnp.dot(a_ref[...], w_ref[...],
                          preferred_element_type=jnp.float32) + b_ref[...])


def _mm_bias(a, w, b, blk):
    m, k = a.shape
    n = w.shape[1]
    return pl.pallas_call(
        _mm_bias_body,
        grid=(m // blk,),
        in_specs=[pl.BlockSpec((blk, k), lambda i: (i, 0)),
                  pl.BlockSpec((k, n), lambda i: (0, 0)),
                  pl.BlockSpec((1, n), lambda i: (0, 0))],
        out_specs=pl.BlockSpec((blk, n), lambda i: (i, 0)),
        out_shape=jax.ShapeDtypeStruct((m, n), jnp.float32),
    )(a, w, b.reshape(1, n))


def _mlp_body(h_ref, p_ref, w1_ref, b1_ref, w2_ref, b2_ref, gs_ref, bt_ref,
              eps_ref, o_ref):
    h = h_ref[...]
    a = h * (1.0 + eps_ref[0, 0]) + p_ref[0] + p_ref[1]
    t = jnp.maximum(jnp.dot(a, w1_ref[...],
                            preferred_element_type=jnp.float32) + b1_ref[...],
                    0.0)
    t = jnp.dot(t, w2_ref[...], preferred_element_type=jnp.float32) + b2_ref[...]
    t = t * gs_ref[...] + bt_ref[...]
    o_ref[...] = jnp.maximum(t, 0.0) + h


def _mlp(h, parts, w1, b1, w2, b2, gscale, beta, eps, blk=2000):
    full = lambda i: (0, 0)
    return pl.pallas_call(
        _mlp_body,
        grid=(_N // blk,),
        in_specs=[pl.BlockSpec((blk, _D), lambda i: (i, 0)),
                  pl.BlockSpec((2, blk, _D), lambda i: (0, i, 0)),
                  pl.BlockSpec((_D, _D), full),
                  pl.BlockSpec((1, _D), full),
                  pl.BlockSpec((_D, _D), full),
                  pl.BlockSpec((1, _D), full),
                  pl.BlockSpec((1, _D), full),
                  pl.BlockSpec((1, _D), full),
                  pl.BlockSpec((1, 1), full)],
        out_specs=pl.BlockSpec((blk, _D), lambda i: (i, 0)),
        out_shape=jax.ShapeDtypeStruct((_N, _D), jnp.float32),
    )(h, parts, w1, b1.reshape(1, _D), w2, b2.reshape(1, _D),
      gscale.reshape(1, _D), beta.reshape(1, _D), eps.reshape(1, 1))


def _head_body(h_ref, b_ref, gx_ref, wg_ref, bg_ref, wc_ref, bc_ref, o_ref):
    g_iota = lax.broadcasted_iota(jnp.int32, (1, 16), 1)
    oh = (b_ref[...] == g_iota).astype(jnp.float32)
    dn = (((0,), (0,)), ((), ()))
    sums = lax.dot_general(oh, h_ref[...], dn,
                           preferred_element_type=jnp.float32)
    ones = jnp.ones((_N, 1), jnp.float32)
    counts = lax.dot_general(oh, ones, dn,
                             preferred_element_type=jnp.float32)
    pooled = sums / jnp.maximum(counts, 1.0)
    g = jnp.maximum(jnp.dot(gx_ref[...], wg_ref[...],
                            preferred_element_type=jnp.float32) + bg_ref[...],
                    0.0)
    out = (jnp.dot(pooled, wc_ref[0:_D, :],
                   preferred_element_type=jnp.float32)
           + jnp.dot(g, wc_ref[_D:_D + 32, :],
                     preferred_element_type=jnp.float32)
           + bc_ref[...])
    o_ref[...] = out


def _head(h, batch2d, gx, wg, bg, wc, bc):
    return pl.pallas_call(
        _head_body,
        out_shape=jax.ShapeDtypeStruct((16, _D), jnp.float32),
    )(h, batch2d, gx, wg, bg.reshape(1, -1), wc, bc.reshape(1, -1))


def kernel(x, edge_index, edge_attr, batch, global_x, params):
    src = edge_index[0]
    dst = edge_index[1]
    h = _mm_bias(x, params['W_in'], params['b_in'], 2000)
    e = _mm_bias(edge_attr, params['W_e'], params['b_e'], 8000)
    bn_scale = 1.0 / jnp.sqrt(jnp.float32(1.0 + 1e-5))
    for i in range(3):
        p = params['layers'][i]
        parts = _sc_edge(h, e, src, dst).reshape(2, _N, _D)
        h = _mlp(h, parts, p['W1'], p['b1'], p['W2'], p['b2'],
                 p['gamma'] * bn_scale, p['beta'], p['eps'])
    return _head(h, batch.reshape(_N, 1), global_x,
                 params['Wg'], params['bg'], params['Wc'], params['bc'])


# speedup vs baseline: 1.5711x; 1.2030x over previous
import functools

import jax
import jax.numpy as jnp
from jax import lax
from jax.experimental import pallas as pl
from jax.experimental.pallas import tpu as pltpu
from jax.experimental.pallas import tpu_sc as plsc

_N = 10000
_E = 320000
_D = 128

_NC = 2
_NS = 16
_NW = _NC * _NS
_EPT = _E // _NW
_C = 80
_NCH = _EPT // _C
_RB = 80
_NRB = _N // _RB
_ZB = 8
_NI = 4
_NM = 3
_UN = 12

_SC_MESH = plsc.VectorSubcoreMesh(core_axis_name="c", subcore_axis_name="s")


@functools.partial(
    pl.kernel,
    out_type=jax.ShapeDtypeStruct((2 * _N, _D), jnp.float32),
    mesh=_SC_MESH,
    scratch_types=(
        [pltpu.VMEM((_C,), jnp.int32)] * (2 * _NI)
        + [pltpu.VMEM((_C, _D), jnp.float32)] * _NM
        + [pltpu.VMEM((_ZB, _D), jnp.float32)]
        + [pltpu.VMEM_SHARED((_N, _D), jnp.float32)]
        + [pltpu.SemaphoreType.DMA] * (_NI + 3 * _NM)
    ),
)
def _sc_edge(h_hbm, e_hbm, src_hbm, dst_hbm, out_hbm,
             si0, si1, si2, si3, di0, di1, di2, di3, m0, m1, m2, zbuf, agg,
             i0, i1, i2, i3, e0, e1, e2, g0, g1, g2, s0, s1, s2):
    sidx = (si0, si1, si2, si3)
    didx = (di0, di1, di2, di3)
    msg = (m0, m1, m2)
    isem = (i0, i1, i2, i3)
    esem = (e0, e1, e2)
    gsem = (g0, g1, g2)
    ssem = (s0, s1, s2)
    c = lax.axis_index("c")
    s = lax.axis_index("s")
    wid = s * _NC + c

    blo = s * _NRB // _NS
    bhi = (s + 1) * _NRB // _NS

    def _z(i, t):
        for j in range(_D // 16):
            zbuf[i, pl.ds(j * 16, 16)] = jnp.zeros((16,), jnp.float32)
        return t
    lax.fori_loop(0, _ZB, _z, 0)

    def _zb(b, t):
        pltpu.sync_copy(zbuf, agg.at[pl.ds(b * _ZB, _ZB)])
        return t
    lax.fori_loop(blo * (_RB // _ZB), bhi * (_RB // _ZB), _zb, 0)

    def _stage_idx(n, bi):
        base = wid * _EPT + n * _C
        pltpu.async_copy(src_hbm.at[pl.ds(base, _C)], sidx[bi], isem[bi])
        pltpu.async_copy(dst_hbm.at[pl.ds(base, _C)], didx[bi], isem[bi])

    def _wait_idx(n, bi):
        base = wid * _EPT + n * _C
        pltpu.make_async_copy(src_hbm.at[pl.ds(base, _C)], sidx[bi],
                              isem[bi]).wait()
        pltpu.make_async_copy(dst_hbm.at[pl.ds(base, _C)], didx[bi],
                              isem[bi]).wait()

    def _stage_e(n, b):
        base = wid * _EPT + n * _C
        pltpu.async_copy(e_hbm.at[pl.ds(base, _C)], msg[b], esem[b])

    def _wait_e(n, b):
        base = wid * _EPT + n * _C
        pltpu.make_async_copy(e_hbm.at[pl.ds(base, _C)], msg[b],
                              esem[b]).wait()

    def _gather_add(b, bi):
        pltpu.async_copy(h_hbm.at[sidx[bi]], msg[b], gsem[b], add=True)

    def _wait_gather(b, bi):
        pltpu.make_async_copy(h_hbm.at[sidx[bi]], msg[b], gsem[b]).wait()

    def _wait_scatter(b, bi):
        pltpu.make_async_copy(msg[b], agg.at[didx[bi]], ssem[b]).wait()

    def _process(n, b, bi):
        _wait_gather(b, bi)

        def _v(r, u):
            for j in range(_D // 16):
                sl = pl.ds(j * 16, 16)
                msg[b][r, sl] = jnp.maximum(msg[b][r, sl], 0.0)
            return u
        lax.fori_loop(0, _C, _v, 0)
        pltpu.async_copy(msg[b], agg.at[didx[bi]], ssem[b], add=True)

    plsc.subcore_barrier()

    _stage_idx(0, 0)
    _stage_idx(1, 1)
    _stage_idx(2, 2)
    _stage_e(0, 0)
    _stage_e(1, 1)
    _wait_idx(0, 0)
    _wait_e(0, 0)
    _gather_add(0, 0)

    def _step(n, db):
        @pl.when((n >= 1) & (n + 2 < _NCH))
        def _():
            _wait_scatter((db + 2) % _NM, (db + 3) % _NI)

        @pl.when(n + 3 < _NCH)
        def _():
            _stage_idx(n + 3, (db + 3) % _NI)

        @pl.when(n + 2 < _NCH)
        def _():
            _stage_e(n + 2, (db + 2) % _NM)

        @pl.when(n + 1 < _NCH)
        def _():
            _wait_idx(n + 1, (db + 1) % _NI)
            _wait_e(n + 1, (db + 1) % _NM)
            _gather_add((db + 1) % _NM, (db + 1) % _NI)

        _process(n, db % _NM, db % _NI)

    def _main(i, t):
        for db in range(_UN):
            _step(i * _UN + db, db)
        return t
    lax.fori_loop(0, _NCH // _UN, _main, 0)
    for n in range(_NCH - _NCH % _UN, _NCH):
        _step(n, n % _UN)
    for n in range(_NCH - 3, _NCH):
        _wait_scatter(n % _NM, n % _NI)

    plsc.subcore_barrier()

    def _wb(b, t):
        pltpu.sync_copy(agg.at[pl.ds(b * _RB, _RB)],
                        out_hbm.at[pl.ds(c * _N + b * _RB, _RB)])
        return t
    lax.fori_loop(blo, bhi, _wb, 0)


def _mm_bias_body(a_ref, w_ref, b_ref, o_ref):
    o_ref[...] = (jnp.dot(a_ref[...], w_ref[...],
                          preferred_element_type=jnp.float32) + b_ref[...])


def _mm_bias(a, w, b, blk):
    m, k = a.shape
    n = w.shape[1]
    return pl.pallas_call(
        _mm_bias_body,
        grid=(m // blk,),
        in_specs=[pl.BlockSpec((blk, k), lambda i: (i, 0)),
                  pl.BlockSpec((k, n), lambda i: (0, 0)),
                  pl.BlockSpec((1, n), lambda i: (0, 0))],
        out_specs=pl.BlockSpec((blk, n), lambda i: (i, 0)),
        out_shape=jax.ShapeDtypeStruct((m, n), jnp.float32),
    )(a, w, b.reshape(1, n))


def _mlp_body(h_ref, p_ref, w1_ref, b1_ref, w2_ref, b2_ref, gs_ref, bt_ref,
              eps_ref, o_ref):
    h = h_ref[...]
    a = h * (1.0 + eps_ref[0, 0]) + p_ref[0] + p_ref[1]
    t = jnp.maximum(jnp.dot(a, w1_ref[...],
                            preferred_element_type=jnp.float32) + b1_ref[...],
                    0.0)
    t = jnp.dot(t, w2_ref[...], preferred_element_type=jnp.float32) + b2_ref[...]
    t = t * gs_ref[...] + bt_ref[...]
    o_ref[...] = jnp.maximum(t, 0.0) + h


def _mlp(h, parts, w1, b1, w2, b2, gscale, beta, eps, blk=2000):
    full = lambda i: (0, 0)
    return pl.pallas_call(
        _mlp_body,
        grid=(_N // blk,),
        in_specs=[pl.BlockSpec((blk, _D), lambda i: (i, 0)),
                  pl.BlockSpec((2, blk, _D), lambda i: (0, i, 0)),
                  pl.BlockSpec((_D, _D), full),
                  pl.BlockSpec((1, _D), full),
                  pl.BlockSpec((_D, _D), full),
                  pl.BlockSpec((1, _D), full),
                  pl.BlockSpec((1, _D), full),
                  pl.BlockSpec((1, _D), full),
                  pl.BlockSpec((1, 1), full)],
        out_specs=pl.BlockSpec((blk, _D), lambda i: (i, 0)),
        out_shape=jax.ShapeDtypeStruct((_N, _D), jnp.float32),
    )(h, parts, w1, b1.reshape(1, _D), w2, b2.reshape(1, _D),
      gscale.reshape(1, _D), beta.reshape(1, _D), eps.reshape(1, 1))


def _head_body(h_ref, b_ref, gx_ref, wg_ref, bg_ref, wc_ref, bc_ref, o_ref):
    g_iota = lax.broadcasted_iota(jnp.int32, (1, 16), 1)
    oh = (b_ref[...] == g_iota).astype(jnp.float32)
    dn = (((0,), (0,)), ((), ()))
    sums = lax.dot_general(oh, h_ref[...], dn,
                           preferred_element_type=jnp.float32)
    ones = jnp.ones((_N, 1), jnp.float32)
    counts = lax.dot_general(oh, ones, dn,
                             preferred_element_type=jnp.float32)
    pooled = sums / jnp.maximum(counts, 1.0)
    g = jnp.maximum(jnp.dot(gx_ref[...], wg_ref[...],
                            preferred_element_type=jnp.float32) + bg_ref[...],
                    0.0)
    out = (jnp.dot(pooled, wc_ref[0:_D, :],
                   preferred_element_type=jnp.float32)
           + jnp.dot(g, wc_ref[_D:_D + 32, :],
                     preferred_element_type=jnp.float32)
           + bc_ref[...])
    o_ref[...] = out


def _head(h, batch2d, gx, wg, bg, wc, bc):
    return pl.pallas_call(
        _head_body,
        out_shape=jax.ShapeDtypeStruct((16, _D), jnp.float32),
    )(h, batch2d, gx, wg, bg.reshape(1, -1), wc, bc.reshape(1, -1))


def kernel(x, edge_index, edge_attr, batch, global_x, params):
    src = edge_index[0]
    dst = edge_index[1]
    h = _mm_bias(x, params['W_in'], params['b_in'], 2000)
    e = _mm_bias(edge_attr, params['W_e'], params['b_e'], 8000)
    bn_scale = 1.0 / jnp.sqrt(jnp.float32(1.0 + 1e-5))
    for i in range(3):
        p = params['layers'][i]
        parts = _sc_edge(h, e, src, dst).reshape(2, _N, _D)
        h = _mlp(h, parts, p['W1'], p['b1'], p['W2'], p['b2'],
                 p['gamma'] * bn_scale, p['beta'], p['eps'])
    return _head(h, batch.reshape(_N, 1), global_x,
                 params['Wg'], params['bg'], params['Wc'], params['bc'])


# speedup vs baseline: 1.5976x; 1.0169x over previous
import functools

import jax
import jax.numpy as jnp
from jax import lax
from jax.experimental import pallas as pl
from jax.experimental.pallas import tpu as pltpu
from jax.experimental.pallas import tpu_sc as plsc

_N = 10000
_E = 320000
_D = 128

_NC = 2
_NS = 16
_NW = _NC * _NS
_EPT = _E // _NW
_C = 80
_NCH = _EPT // _C
_RB = 80
_NRB = _N // _RB
_ZB = 8
_NI = 4
_NM = 3
_UN = 12

_SC_MESH = plsc.VectorSubcoreMesh(core_axis_name="c", subcore_axis_name="s")


@functools.partial(
    pl.kernel,
    out_type=jax.ShapeDtypeStruct((2 * _N, _D), jnp.float32),
    mesh=_SC_MESH,
    scratch_types=(
        [pltpu.VMEM((_C,), jnp.int32)] * (2 * _NI)
        + [pltpu.VMEM((_C, _D), jnp.float32)] * _NM
        + [pltpu.VMEM((_ZB, _D), jnp.float32)]
        + [pltpu.VMEM_SHARED((_N, _D), jnp.float32)]
        + [pltpu.SemaphoreType.DMA] * (_NI + 3 * _NM + 1)
    ),
)
def _sc_edge(h_hbm, e_hbm, src_hbm, dst_hbm, out_hbm,
             si0, si1, si2, si3, di0, di1, di2, di3, m0, m1, m2, zbuf, agg,
             i0, i1, i2, i3, e0, e1, e2, g0, g1, g2, s0, s1, s2, zsem):
    sidx = (si0, si1, si2, si3)
    didx = (di0, di1, di2, di3)
    msg = (m0, m1, m2)
    isem = (i0, i1, i2, i3)
    esem = (e0, e1, e2)
    gsem = (g0, g1, g2)
    ssem = (s0, s1, s2)
    c = lax.axis_index("c")
    s = lax.axis_index("s")
    wid = s * _NC + c

    blo = s * _NRB // _NS
    bhi = (s + 1) * _NRB // _NS

    def _z(i, t):
        for j in range(_D // 16):
            zbuf[i, pl.ds(j * 16, 16)] = jnp.zeros((16,), jnp.float32)
        return t
    lax.fori_loop(0, _ZB, _z, 0)

    def _zb(b, t):
        pltpu.async_copy(zbuf, agg.at[pl.ds(b * _ZB, _ZB)], zsem)
        return t
    lax.fori_loop(blo * (_RB // _ZB), bhi * (_RB // _ZB), _zb, 0)

    def _zw(b, t):
        pltpu.make_async_copy(zbuf, agg.at[pl.ds(b * _ZB, _ZB)], zsem).wait()
        return t
    lax.fori_loop(blo * (_RB // _ZB), bhi * (_RB // _ZB), _zw, 0)

    def _stage_idx(n, bi):
        base = wid * _EPT + n * _C
        pltpu.async_copy(src_hbm.at[pl.ds(base, _C)], sidx[bi], isem[bi])
        pltpu.async_copy(dst_hbm.at[pl.ds(base, _C)], didx[bi], isem[bi])

    def _wait_idx(n, bi):
        base = wid * _EPT + n * _C
        pltpu.make_async_copy(src_hbm.at[pl.ds(base, _C)], sidx[bi],
                              isem[bi]).wait()
        pltpu.make_async_copy(dst_hbm.at[pl.ds(base, _C)], didx[bi],
                              isem[bi]).wait()

    def _stage_e(n, b):
        base = wid * _EPT + n * _C
        pltpu.async_copy(e_hbm.at[pl.ds(base, _C)], msg[b], esem[b])

    def _wait_e(n, b):
        base = wid * _EPT + n * _C
        pltpu.make_async_copy(e_hbm.at[pl.ds(base, _C)], msg[b],
                              esem[b]).wait()

    def _gather_add(b, bi):
        pltpu.async_copy(h_hbm.at[sidx[bi]], msg[b], gsem[b], add=True)

    def _wait_gather(b, bi):
        pltpu.make_async_copy(h_hbm.at[sidx[bi]], msg[b], gsem[b]).wait()

    def _wait_scatter(b, bi):
        pltpu.make_async_copy(msg[b], agg.at[didx[bi]], ssem[b]).wait()

    def _process(n, b, bi):
        _wait_gather(b, bi)

        def _v(r, u):
            for j in range(_D // 16):
                sl = pl.ds(j * 16, 16)
                msg[b][r, sl] = jnp.maximum(msg[b][r, sl], 0.0)
            return u
        lax.fori_loop(0, _C, _v, 0)
        pltpu.async_copy(msg[b], agg.at[didx[bi]], ssem[b], add=True)

    plsc.subcore_barrier()

    _stage_idx(0, 0)
    _stage_idx(1, 1)
    _stage_idx(2, 2)
    _stage_e(0, 0)
    _stage_e(1, 1)
    _wait_idx(0, 0)
    _wait_e(0, 0)
    _gather_add(0, 0)

    def _step(n, db):
        @pl.when((n >= 1) & (n + 2 < _NCH))
        def _():
            _wait_scatter((db + 2) % _NM, (db + 3) % _NI)

        @pl.when(n + 3 < _NCH)
        def _():
            _stage_idx(n + 3, (db + 3) % _NI)

        @pl.when(n + 2 < _NCH)
        def _():
            _stage_e(n + 2, (db + 2) % _NM)

        @pl.when(n + 1 < _NCH)
        def _():
            _wait_idx(n + 1, (db + 1) % _NI)
            _wait_e(n + 1, (db + 1) % _NM)
            _gather_add((db + 1) % _NM, (db + 1) % _NI)

        _process(n, db % _NM, db % _NI)

    def _main(i, t):
        for db in range(_UN):
            _step(i * _UN + db, db)
        return t
    lax.fori_loop(0, _NCH // _UN, _main, 0)
    for n in range(_NCH - _NCH % _UN, _NCH):
        _step(n, n % _UN)
    for n in range(_NCH - 3, _NCH):
        _wait_scatter(n % _NM, n % _NI)

    plsc.subcore_barrier()

    def _wb(b, t):
        pltpu.sync_copy(agg.at[pl.ds(b * _RB, _RB)],
                        out_hbm.at[pl.ds(c * _N + b * _RB, _RB)])
        return t
    lax.fori_loop(blo, bhi, _wb, 0)


def _mm_bias_body(a_ref, w_ref, b_ref, o_ref):
    o_ref[...] = (jnp.dot(a_ref[...], w_ref[...],
                          preferred_element_type=jnp.float32) + b_ref[...])


def _mm_bias(a, w, b, blk):
    m, k = a.shape
    n = w.shape[1]
    return pl.pallas_call(
        _mm_bias_body,
        grid=(m // blk,),
        in_specs=[pl.BlockSpec((blk, k), lambda i: (i, 0)),
                  pl.BlockSpec((k, n), lambda i: (0, 0)),
                  pl.BlockSpec((1, n), lambda i: (0, 0))],
        out_specs=pl.BlockSpec((blk, n), lambda i: (i, 0)),
        out_shape=jax.ShapeDtypeStruct((m, n), jnp.float32),
    )(a, w, b.reshape(1, n))


def _mlp_body(h_ref, p_ref, w1_ref, b1_ref, w2_ref, b2_ref, gs_ref, bt_ref,
              eps_ref, o_ref):
    h = h_ref[...]
    a = h * (1.0 + eps_ref[0, 0]) + p_ref[0] + p_ref[1]
    t = jnp.maximum(jnp.dot(a, w1_ref[...],
                            preferred_element_type=jnp.float32) + b1_ref[...],
                    0.0)
    t = jnp.dot(t, w2_ref[...], preferred_element_type=jnp.float32) + b2_ref[...]
    t = t * gs_ref[...] + bt_ref[...]
    o_ref[...] = jnp.maximum(t, 0.0) + h


def _mlp(h, parts, w1, b1, w2, b2, gscale, beta, eps, blk=2000):
    full = lambda i: (0, 0)
    return pl.pallas_call(
        _mlp_body,
        grid=(_N // blk,),
        in_specs=[pl.BlockSpec((blk, _D), lambda i: (i, 0)),
                  pl.BlockSpec((2, blk, _D), lambda i: (0, i, 0)),
                  pl.BlockSpec((_D, _D), full),
                  pl.BlockSpec((1, _D), full),
                  pl.BlockSpec((_D, _D), full),
                  pl.BlockSpec((1, _D), full),
                  pl.BlockSpec((1, _D), full),
                  pl.BlockSpec((1, _D), full),
                  pl.BlockSpec((1, 1), full)],
        out_specs=pl.BlockSpec((blk, _D), lambda i: (i, 0)),
        out_shape=jax.ShapeDtypeStruct((_N, _D), jnp.float32),
    )(h, parts, w1, b1.reshape(1, _D), w2, b2.reshape(1, _D),
      gscale.reshape(1, _D), beta.reshape(1, _D), eps.reshape(1, 1))


def _head_body(h_ref, b_ref, gx_ref, wg_ref, bg_ref, wc_ref, bc_ref, o_ref):
    g_iota = lax.broadcasted_iota(jnp.int32, (1, 16), 1)
    oh = (b_ref[...] == g_iota).astype(jnp.float32)
    dn = (((0,), (0,)), ((), ()))
    sums = lax.dot_general(oh, h_ref[...], dn,
                           preferred_element_type=jnp.float32)
    ones = jnp.ones((_N, 1), jnp.float32)
    counts = lax.dot_general(oh, ones, dn,
                             preferred_element_type=jnp.float32)
    pooled = sums / jnp.maximum(counts, 1.0)
    g = jnp.maximum(jnp.dot(gx_ref[...], wg_ref[...],
                            preferred_element_type=jnp.float32) + bg_ref[...],
                    0.0)
    out = (jnp.dot(pooled, wc_ref[0:_D, :],
                   preferred_element_type=jnp.float32)
           + jnp.dot(g, wc_ref[_D:_D + 32, :],
                     preferred_element_type=jnp.float32)
           + bc_ref[...])
    o_ref[...] = out


def _head(h, batch2d, gx, wg, bg, wc, bc):
    return pl.pallas_call(
        _head_body,
        out_shape=jax.ShapeDtypeStruct((16, _D), jnp.float32),
    )(h, batch2d, gx, wg, bg.reshape(1, -1), wc, bc.reshape(1, -1))


def kernel(x, edge_index, edge_attr, batch, global_x, params):
    src = edge_index[0]
    dst = edge_index[1]
    h = _mm_bias(x, params['W_in'], params['b_in'], 2000)
    e = _mm_bias(edge_attr, params['W_e'], params['b_e'], 8000)
    bn_scale = 1.0 / jnp.sqrt(jnp.float32(1.0 + 1e-5))
    for i in range(3):
        p = params['layers'][i]
        parts = _sc_edge(h, e, src, dst).reshape(2, _N, _D)
        h = _mlp(h, parts, p['W1'], p['b1'], p['W2'], p['b2'],
                 p['gamma'] * bn_scale, p['beta'], p['eps'])
    return _head(h, batch.reshape(_N, 1), global_x,
                 params['Wg'], params['bg'], params['Wc'], params['bc'])
